# baseline (speedup 1.0000x reference)
import functools

import jax
import jax.numpy as jnp
from jax import lax
from jax.experimental import pallas as pl
from jax.experimental.pallas import tpu as pltpu
from jax.experimental.pallas import tpu_sc as plsc

N = 10000
D = 128
E = 320000
NC = 2
NS = 16
NW = NC * NS
EPW = E // NW
CHUNK = 64
NCHUNK = 160
EPAD = NCHUNK * CHUNK
NROUND = NCHUNK // 4
NA = N + 8
RPT = N // NS
RZ = 125

_mesh = plsc.VectorSubcoreMesh(
    core_axis_name="c", subcore_axis_name="s", num_cores=NC, num_subcores=NS
)


@functools.partial(
    pl.kernel,
    out_type=jax.ShapeDtypeStruct((NC, NS, RPT, D), jnp.float32),
    mesh=_mesh,
    scratch_types=[
        pltpu.VMEM((CHUNK,), jnp.int32),
        pltpu.VMEM((CHUNK,), jnp.int32),
        pltpu.VMEM((CHUNK,), jnp.int32),
        pltpu.VMEM((CHUNK,), jnp.int32),
        pltpu.VMEM((CHUNK, D), jnp.float32),
        pltpu.VMEM_SHARED((NA, D), jnp.float32),
        pltpu.SemaphoreType.DMA,
        pltpu.SemaphoreType.DMA,
        pltpu.SemaphoreType.DMA,
        pltpu.SemaphoreType.DMA,
    ],
)
def _sc_degree(dst_hbm, out_hbm, i0, i1, i2, i3, ones_v, acc, s0, s1, s2, s3):
    c = lax.axis_index("c")
    s = lax.axis_index("s")
    wid = c * NS + s
    idxs = (i0, i1, i2, i3)
    ssem = (s0, s1, s2, s3)

    def zrow(i, _):
        for k in range(D // 16):
            ones_v[i, pl.ds(k * 16, 16)] = jnp.zeros((16,), jnp.float32)
        return 0

    lax.fori_loop(0, CHUNK, zrow, 0)
    for r in range(RPT // CHUNK):
        pltpu.sync_copy(ones_v, acc.at[pl.ds(s * RPT + r * CHUNK, CHUNK)])
    rtail = RPT % CHUNK
    pltpu.sync_copy(
        ones_v.at[pl.ds(0, rtail)],
        acc.at[pl.ds(s * RPT + RPT - rtail, rtail)],
    )

    def orow(i, _):
        for k in range(D // 16):
            ones_v[i, pl.ds(k * 16, 16)] = jnp.ones((16,), jnp.float32)
        return 0

    lax.fori_loop(0, CHUNK, orow, 0)
    plsc.subcore_barrier()

    def load_scatter(b, j):
        pltpu.sync_copy(dst_hbm.at[wid, j, 1], idxs[b])
        pltpu.async_copy(ones_v, acc.at[idxs[b]], ssem[b], add=True)

    def wait_scatter(b):
        pltpu.make_async_copy(ones_v, acc.at[idxs[b]], ssem[b]).wait()

    for b in range(4):
        load_scatter(b, b)

    def chunk(jj, _):
        base = 4 * jj
        for b in range(4):
            wait_scatter(b)
            load_scatter(b, base + b)
        return 0

    lax.fori_loop(1, NROUND, chunk, 0)
    for b in range(4):
        wait_scatter(b)
    plsc.subcore_barrier()
    pltpu.sync_copy(acc.at[pl.ds(s * RPT, RPT)], out_hbm.at[c, s])


@functools.partial(
    pl.kernel,
    out_type=jax.ShapeDtypeStruct((NC, NS, RPT, D), jnp.float32),
    mesh=_mesh,
    scratch_types=[
        pltpu.VMEM((2, CHUNK), jnp.int32),
        pltpu.VMEM((2, CHUNK), jnp.int32),
        pltpu.VMEM((2, CHUNK), jnp.int32),
        pltpu.VMEM((2, CHUNK), jnp.int32),
        pltpu.VMEM((CHUNK, D), jnp.float32),
        pltpu.VMEM((CHUNK, D), jnp.float32),
        pltpu.VMEM((CHUNK, D), jnp.float32),
        pltpu.VMEM((CHUNK, D), jnp.float32),
        pltpu.VMEM_SHARED((NA, D), jnp.float32),
        pltpu.SemaphoreType.DMA,
        pltpu.SemaphoreType.DMA,
        pltpu.SemaphoreType.DMA,
        pltpu.SemaphoreType.DMA,
        pltpu.SemaphoreType.DMA,
        pltpu.SemaphoreType.DMA,
        pltpu.SemaphoreType.DMA,
        pltpu.SemaphoreType.DMA,
    ],
)
def _sc_aggregate(
    table_hbm, idx_hbm, out_hbm, i0, i1, i2, i3, r0, r1, r2, r3, acc,
    g0, g1, g2, g3, s0, s1, s2, s3,
):
    c = lax.axis_index("c")
    s = lax.axis_index("s")
    wid = c * NS + s
    idxs = (i0, i1, i2, i3)
    rows = (r0, r1, r2, r3)
    gsem = (g0, g1, g2, g3)
    ssem = (s0, s1, s2, s3)

    def zrow(i, _):
        for k in range(D // 16):
            r0[i, pl.ds(k * 16, 16)] = jnp.zeros((16,), jnp.float32)
        return 0

    lax.fori_loop(0, CHUNK, zrow, 0)
    for r in range(RPT // CHUNK):
        pltpu.sync_copy(r0, acc.at[pl.ds(s * RPT + r * CHUNK, CHUNK)])
    rtail = RPT % CHUNK
    pltpu.sync_copy(
        r0.at[pl.ds(0, rtail)], acc.at[pl.ds(s * RPT + RPT - rtail, rtail)]
    )
    plsc.subcore_barrier()

    def load_gather(b, j):
        pltpu.sync_copy(idx_hbm.at[wid, j], idxs[b])
        pltpu.async_copy(table_hbm.at[idxs[b].at[0]], rows[b], gsem[b])

    def wait_gather(b):
        pltpu.make_async_copy(
            table_hbm.at[idxs[b].at[0]], rows[b], gsem[b]
        ).wait()

    def start_scatter(b):
        pltpu.async_copy(rows[b], acc.at[idxs[b].at[1]], ssem[b], add=True)

    def wait_scatter(b):
        pltpu.make_async_copy(
            rows[b], acc.at[idxs[b].at[1]], ssem[b]
        ).wait()

    load_gather(0, 0)
    load_gather(1, 1)
    load_gather(2, 2)
    wait_gather(0)
    start_scatter(0)
    load_gather(3, 3)
    wait_gather(1)
    start_scatter(1)

    def step(jj, _):
        base = 4 * jj
        for p in range(4):
            wait_scatter(p)
            load_gather(p, base + p)
            b2 = (p + 2) % 4
            wait_gather(b2)
            start_scatter(b2)
        return 0

    lax.fori_loop(1, NROUND, step, 0)
    wait_gather(2)
    start_scatter(2)
    wait_gather(3)
    start_scatter(3)
    for b in range(4):
        wait_scatter(b)
    plsc.subcore_barrier()
    pltpu.sync_copy(acc.at[pl.ds(s * RPT, RPT)], out_hbm.at[c, s])


def _tc_matmul(x_ref, w_ref, out_ref):
    out_ref[...] = jnp.dot(
        x_ref[...], w_ref[...], preferred_element_type=jnp.float32
    )


def _tc_scale(hist_ref, h_ref, out_ref):
    dinv = lax.rsqrt(1.0 + hist_ref[0] + hist_ref[1])
    out_ref[...] = dinv * h_ref[...]


def _tc_mid(hist_ref, agg_ref, hs_ref, b_ref, w_ref, out_ref):
    dinv = lax.rsqrt(1.0 + hist_ref[0] + hist_ref[1])
    pre = dinv * (agg_ref[0] + agg_ref[1] + hs_ref[...]) + b_ref[...]
    h1 = jnp.maximum(pre, 0.0)
    out_ref[...] = dinv * jnp.dot(
        h1, w_ref[...], preferred_element_type=jnp.float32
    )


def _tc_final(hist_ref, agg_ref, hs_ref, b_ref, out_ref):
    dinv = lax.rsqrt(1.0 + hist_ref[0] + hist_ref[1])
    out_ref[...] = dinv * (agg_ref[0] + agg_ref[1] + hs_ref[...]) + b_ref[...]


_f32 = functools.partial(jax.ShapeDtypeStruct, dtype=jnp.float32)


@jax.jit
def kernel(x, edge_index, W1, b1, W2, b2):
    pad_src = jnp.zeros((NW, EPAD - EPW), jnp.int32)
    pad_dst = jnp.full((NW, EPAD - EPW), N, jnp.int32)
    src = jnp.concatenate(
        [edge_index[0].reshape(NW, EPW), pad_src], axis=1
    ).reshape(NW, NCHUNK, 1, CHUNK)
    dst = jnp.concatenate(
        [edge_index[1].reshape(NW, EPW), pad_dst], axis=1
    ).reshape(NW, NCHUNK, 1, CHUNK)
    idx = jnp.concatenate([src, dst], axis=2)

    hist = _sc_degree(idx).reshape(NC, N, D)[:, :, 0:1]
    h1 = pl.pallas_call(_tc_matmul, out_shape=_f32((N, D)))(x, W1)

    hs1 = pl.pallas_call(_tc_scale, out_shape=_f32((N, D)))(hist, h1)

    agg1 = _sc_aggregate(hs1, idx).reshape(NC, N, D)

    hs2 = pl.pallas_call(_tc_mid, out_shape=_f32((N, D)))(
        hist, agg1, hs1, b1.reshape(1, D), W2
    )

    agg2 = _sc_aggregate(hs2, idx).reshape(NC, N, D)

    out = pl.pallas_call(_tc_final, out_shape=_f32((N, D)))(
        hist, agg2, hs2, b2.reshape(1, D)
    )
    return out


# speedup vs baseline: 1.0247x; 1.0247x over previous
import functools

import jax
import jax.numpy as jnp
from jax import lax
from jax.experimental import pallas as pl
from jax.experimental.pallas import tpu as pltpu
from jax.experimental.pallas import tpu_sc as plsc

N = 10000
D = 128
E = 320000
NC = 2
NS = 16
NW = NC * NS
EPW = E // NW
CHUNK = 128
NCHUNK = 80
EPAD = NCHUNK * CHUNK
G = 40
NG = NCHUNK // G
NA = N + 8
RPT = N // NS
RZ = 125

_mesh = plsc.VectorSubcoreMesh(
    core_axis_name="c", subcore_axis_name="s", num_cores=NC, num_subcores=NS
)


@functools.partial(
    pl.kernel,
    out_type=jax.ShapeDtypeStruct((NC, NS, RPT, D), jnp.float32),
    mesh=_mesh,
    scratch_types=[
        pltpu.VMEM((G, 2, CHUNK), jnp.int32),
        pltpu.VMEM((CHUNK, D), jnp.float32),
        pltpu.VMEM_SHARED((NA, D), jnp.float32),
        pltpu.SemaphoreType.DMA,
        pltpu.SemaphoreType.DMA,
        pltpu.SemaphoreType.DMA,
        pltpu.SemaphoreType.DMA,
    ],
)
def _sc_degree(dst_hbm, out_hbm, idx_g, ones_v, acc, s0, s1, s2, s3):
    c = lax.axis_index("c")
    s = lax.axis_index("s")
    wid = c * NS + s
    ssem = (s0, s1, s2, s3)

    def zrow(i, _):
        for k in range(D // 16):
            ones_v[i, pl.ds(k * 16, 16)] = jnp.zeros((16,), jnp.float32)
        return 0

    lax.fori_loop(0, CHUNK, zrow, 0)
    for r in range(RPT // CHUNK):
        pltpu.sync_copy(ones_v, acc.at[pl.ds(s * RPT + r * CHUNK, CHUNK)])
    rtail = RPT % CHUNK
    pltpu.sync_copy(
        ones_v.at[pl.ds(0, rtail)],
        acc.at[pl.ds(s * RPT + RPT - rtail, rtail)],
    )

    def orow(i, _):
        for k in range(D // 16):
            ones_v[i, pl.ds(k * 16, 16)] = jnp.ones((16,), jnp.float32)
        return 0

    lax.fori_loop(0, CHUNK, orow, 0)
    plsc.subcore_barrier()

    def start_scatter(b, j):
        pltpu.async_copy(ones_v, acc.at[idx_g.at[j, 1]], ssem[b], add=True)

    def wait_scatter(b, j):
        pltpu.make_async_copy(ones_v, acc.at[idx_g.at[j, 1]], ssem[b]).wait()

    for grp in range(NG):
        pltpu.sync_copy(dst_hbm.at[wid, pl.ds(grp * G, G)], idx_g)
        for b in range(4):
            start_scatter(b, b)

        def round4(jj, _):
            base = 4 * jj
            for b in range(4):
                wait_scatter(b, base - 4 + b)
                start_scatter(b, base + b)
            return 0

        lax.fori_loop(1, G // 4, round4, 0)
        for b in range(4):
            wait_scatter(b, G - 4 + b)
    plsc.subcore_barrier()
    pltpu.sync_copy(acc.at[pl.ds(s * RPT, RPT)], out_hbm.at[c, s])


@functools.partial(
    pl.kernel,
    out_type=jax.ShapeDtypeStruct((NC, NS, RPT, D), jnp.float32),
    mesh=_mesh,
    scratch_types=[
        pltpu.VMEM((G, 2, CHUNK), jnp.int32),
        pltpu.VMEM((CHUNK, D), jnp.float32),
        pltpu.VMEM((CHUNK, D), jnp.float32),
        pltpu.VMEM_SHARED((NA, D), jnp.float32),
        pltpu.SemaphoreType.DMA,
        pltpu.SemaphoreType.DMA,
    ],
)
def _sc_aggregate(
    table_hbm, idx_hbm, out_hbm, idx_g, r0, r1, acc, g0, g1
):
    c = lax.axis_index("c")
    s = lax.axis_index("s")
    wid = c * NS + s
    rows = (r0, r1)
    gsem = (g0, g1)

    def zrow(i, _):
        for k in range(D // 16):
            r0[i, pl.ds(k * 16, 16)] = jnp.zeros((16,), jnp.float32)
        return 0

    lax.fori_loop(0, CHUNK, zrow, 0)
    for r in range(RPT // CHUNK):
        pltpu.sync_copy(r0, acc.at[pl.ds(s * RPT + r * CHUNK, CHUNK)])
    rtail = RPT % CHUNK
    pltpu.sync_copy(
        r0.at[pl.ds(0, rtail)], acc.at[pl.ds(s * RPT + RPT - rtail, rtail)]
    )
    plsc.subcore_barrier()

    def start_gather(b, j):
        pltpu.async_copy(table_hbm.at[idx_g.at[j, 0]], rows[b], gsem[b])

    def wait_gather(b, j):
        pltpu.make_async_copy(
            table_hbm.at[idx_g.at[j, 0]], rows[b], gsem[b]
        ).wait()

    def scatter(b, j):
        pltpu.sync_copy(rows[b], acc.at[idx_g.at[j, 1]], add=True)

    for grp in range(NG):
        pltpu.sync_copy(idx_hbm.at[wid, pl.ds(grp * G, G)], idx_g)
        start_gather(0, 0)

        def pair(jj, _):
            j = 2 * jj
            start_gather(1, j + 1)
            wait_gather(0, j)
            scatter(0, j)
            start_gather(0, j + 2)
            wait_gather(1, j + 1)
            scatter(1, j + 1)
            return 0

        lax.fori_loop(0, (G - 2) // 2, pair, 0)
        start_gather(1, G - 1)
        wait_gather(0, G - 2)
        scatter(0, G - 2)
        wait_gather(1, G - 1)
        scatter(1, G - 1)
    plsc.subcore_barrier()
    pltpu.sync_copy(acc.at[pl.ds(s * RPT, RPT)], out_hbm.at[c, s])


def _tc_matmul(x_ref, w_ref, out_ref):
    out_ref[...] = jnp.dot(
        x_ref[...], w_ref[...], preferred_element_type=jnp.float32
    )


def _tc_scale(hist_ref, h_ref, out_ref):
    dinv = lax.rsqrt(1.0 + hist_ref[0] + hist_ref[1])
    out_ref[...] = dinv * h_ref[...]


def _tc_mid(hist_ref, agg_ref, hs_ref, b_ref, w_ref, out_ref):
    dinv = lax.rsqrt(1.0 + hist_ref[0] + hist_ref[1])
    pre = dinv * (agg_ref[0] + agg_ref[1] + hs_ref[...]) + b_ref[...]
    h1 = jnp.maximum(pre, 0.0)
    out_ref[...] = dinv * jnp.dot(
        h1, w_ref[...], preferred_element_type=jnp.float32
    )


def _tc_final(hist_ref, agg_ref, hs_ref, b_ref, out_ref):
    dinv = lax.rsqrt(1.0 + hist_ref[0] + hist_ref[1])
    out_ref[...] = dinv * (agg_ref[0] + agg_ref[1] + hs_ref[...]) + b_ref[...]


_f32 = functools.partial(jax.ShapeDtypeStruct, dtype=jnp.float32)


@jax.jit
def kernel(x, edge_index, W1, b1, W2, b2):
    pad_src = jnp.zeros((NW, EPAD - EPW), jnp.int32)
    pad_dst = jnp.full((NW, EPAD - EPW), N, jnp.int32)
    src = jnp.concatenate(
        [edge_index[0].reshape(NW, EPW), pad_src], axis=1
    ).reshape(NW, NCHUNK, 1, CHUNK)
    dst = jnp.concatenate(
        [edge_index[1].reshape(NW, EPW), pad_dst], axis=1
    ).reshape(NW, NCHUNK, 1, CHUNK)
    idx = jnp.concatenate([src, dst], axis=2)

    hist = _sc_degree(idx).reshape(NC, N, D)[:, :, 0:1]
    h1 = pl.pallas_call(_tc_matmul, out_shape=_f32((N, D)))(x, W1)

    hs1 = pl.pallas_call(_tc_scale, out_shape=_f32((N, D)))(hist, h1)

    agg1 = _sc_aggregate(hs1, idx).reshape(NC, N, D)

    hs2 = pl.pallas_call(_tc_mid, out_shape=_f32((N, D)))(
        hist, agg1, hs1, b1.reshape(1, D), W2
    )

    agg2 = _sc_aggregate(hs2, idx).reshape(NC, N, D)

    out = pl.pallas_call(_tc_final, out_shape=_f32((N, D)))(
        hist, agg2, hs2, b2.reshape(1, D)
    )
    return out


# speedup vs baseline: 1.0318x; 1.0069x over previous
import functools

import jax
import jax.numpy as jnp
from jax import lax
from jax.experimental import pallas as pl
from jax.experimental.pallas import tpu as pltpu
from jax.experimental.pallas import tpu_sc as plsc

N = 10000
D = 128
E = 320000
NC = 2
NS = 16
NW = NC * NS
EPW = E // NW
CHUNK = 128
NCHUNK = 80
EPAD = NCHUNK * CHUNK
NA = N + 8
RPT = N // NS
RZ = 125

_mesh = plsc.VectorSubcoreMesh(
    core_axis_name="c", subcore_axis_name="s", num_cores=NC, num_subcores=NS
)


@functools.partial(
    pl.kernel,
    out_type=jax.ShapeDtypeStruct((NC, NS, RPT, D), jnp.float32),
    mesh=_mesh,
    scratch_types=[
        pltpu.VMEM((2, CHUNK), jnp.int32),
        pltpu.VMEM((2, CHUNK), jnp.int32),
        pltpu.VMEM((2, CHUNK), jnp.int32),
        pltpu.VMEM((2, CHUNK), jnp.int32),
        pltpu.VMEM((CHUNK, D), jnp.float32),
        pltpu.VMEM_SHARED((NA, D), jnp.float32),
        pltpu.SemaphoreType.DMA,
        pltpu.SemaphoreType.DMA,
        pltpu.SemaphoreType.DMA,
        pltpu.SemaphoreType.DMA,
    ],
)
def _sc_degree(dst_hbm, out_hbm, i0, i1, i2, i3, ones_v, acc, s0, s1, s2, s3):
    c = lax.axis_index("c")
    s = lax.axis_index("s")
    wid = c * NS + s
    idxs = (i0, i1, i2, i3)
    ssem = (s0, s1, s2, s3)

    def zrow(i, _):
        for k in range(D // 16):
            ones_v[i, pl.ds(k * 16, 16)] = jnp.zeros((16,), jnp.float32)
        return 0

    lax.fori_loop(0, CHUNK, zrow, 0)
    for r in range(RPT // CHUNK):
        pltpu.sync_copy(ones_v, acc.at[pl.ds(s * RPT + r * CHUNK, CHUNK)])
    rtail = RPT % CHUNK
    pltpu.sync_copy(
        ones_v.at[pl.ds(0, rtail)],
        acc.at[pl.ds(s * RPT + RPT - rtail, rtail)],
    )

    def orow(i, _):
        for k in range(D // 16):
            ones_v[i, pl.ds(k * 16, 16)] = jnp.ones((16,), jnp.float32)
        return 0

    lax.fori_loop(0, CHUNK, orow, 0)
    plsc.subcore_barrier()

    def load_scatter(b, j):
        pltpu.sync_copy(dst_hbm.at[wid, j], idxs[b])
        pltpu.async_copy(ones_v, acc.at[idxs[b].at[1]], ssem[b], add=True)

    def wait_scatter(b):
        pltpu.make_async_copy(ones_v, acc.at[idxs[b].at[1]], ssem[b]).wait()

    for b in range(4):
        load_scatter(b, b)

    def round4(jj, _):
        base = 4 * jj
        for b in range(4):
            wait_scatter(b)
            load_scatter(b, base + b)
        return 0

    lax.fori_loop(1, NCHUNK // 4, round4, 0)
    for b in range(4):
        wait_scatter(b)
    plsc.subcore_barrier()
    pltpu.sync_copy(acc.at[pl.ds(s * RPT, RPT)], out_hbm.at[c, s])


@functools.partial(
    pl.kernel,
    out_type=jax.ShapeDtypeStruct((NC, NS, RPT, D), jnp.float32),
    mesh=_mesh,
    scratch_types=[
        pltpu.VMEM((2, CHUNK), jnp.int32),
        pltpu.VMEM((2, CHUNK), jnp.int32),
        pltpu.VMEM((2, CHUNK), jnp.int32),
        pltpu.VMEM((CHUNK, D), jnp.float32),
        pltpu.VMEM((CHUNK, D), jnp.float32),
        pltpu.VMEM((CHUNK, D), jnp.float32),
        pltpu.VMEM_SHARED((NA, D), jnp.float32),
        pltpu.SemaphoreType.DMA,
        pltpu.SemaphoreType.DMA,
        pltpu.SemaphoreType.DMA,
        pltpu.SemaphoreType.DMA,
        pltpu.SemaphoreType.DMA,
        pltpu.SemaphoreType.DMA,
    ],
)
def _sc_aggregate(
    table_hbm, idx_hbm, out_hbm, i0, i1, i2, r0, r1, r2, acc,
    g0, g1, g2, s0, s1, s2,
):
    c = lax.axis_index("c")
    s = lax.axis_index("s")
    wid = c * NS + s
    idxs = (i0, i1, i2)
    rows = (r0, r1, r2)
    gsem = (g0, g1, g2)
    ssem = (s0, s1, s2)

    def zrow(i, _):
        for k in range(D // 16):
            r0[i, pl.ds(k * 16, 16)] = jnp.zeros((16,), jnp.float32)
        return 0

    lax.fori_loop(0, CHUNK, zrow, 0)
    for r in range(RPT // CHUNK):
        pltpu.sync_copy(r0, acc.at[pl.ds(s * RPT + r * CHUNK, CHUNK)])
    rtail = RPT % CHUNK
    pltpu.sync_copy(
        r0.at[pl.ds(0, rtail)], acc.at[pl.ds(s * RPT + RPT - rtail, rtail)]
    )
    plsc.subcore_barrier()

    def load_gather(b, j):
        pltpu.sync_copy(idx_hbm.at[wid, j], idxs[b])
        pltpu.async_copy(table_hbm.at[idxs[b].at[0]], rows[b], gsem[b])

    def wait_gather(b):
        pltpu.make_async_copy(
            table_hbm.at[idxs[b].at[0]], rows[b], gsem[b]
        ).wait()

    def start_scatter(b):
        pltpu.async_copy(rows[b], acc.at[idxs[b].at[1]], ssem[b], add=True)

    def wait_scatter(b):
        pltpu.make_async_copy(
            rows[b], acc.at[idxs[b].at[1]], ssem[b]
        ).wait()

    load_gather(0, 0)
    load_gather(1, 1)
    wait_gather(0)
    start_scatter(0)
    load_gather(2, 2)
    wait_gather(1)
    start_scatter(1)
    wait_scatter(0)
    load_gather(0, 3)
    wait_gather(2)
    start_scatter(2)
    wait_scatter(1)
    load_gather(1, 4)
    wait_gather(0)
    start_scatter(0)

    def step(jj, _):
        base = 3 * jj + 5
        for p in range(3):
            b = (2 + p) % 3
            wait_scatter(b)
            load_gather(b, base + p)
            bp = (b + 2) % 3
            wait_gather(bp)
            start_scatter(bp)
        return 0

    lax.fori_loop(0, (NCHUNK - 5) // 3, step, 0)
    wait_gather((NCHUNK - 1) % 3)
    start_scatter((NCHUNK - 1) % 3)
    for b in range(3):
        wait_scatter(b)
    plsc.subcore_barrier()
    pltpu.sync_copy(acc.at[pl.ds(s * RPT, RPT)], out_hbm.at[c, s])


def _tc_matmul(x_ref, w_ref, out_ref):
    out_ref[...] = jnp.dot(
        x_ref[...], w_ref[...], preferred_element_type=jnp.float32
    )


def _tc_scale(hist_ref, h_ref, out_ref):
    dinv = lax.rsqrt(1.0 + hist_ref[0] + hist_ref[1])
    out_ref[...] = dinv * h_ref[...]


def _tc_mid(hist_ref, agg_ref, hs_ref, b_ref, w_ref, out_ref):
    dinv = lax.rsqrt(1.0 + hist_ref[0] + hist_ref[1])
    pre = dinv * (agg_ref[0] + agg_ref[1] + hs_ref[...]) + b_ref[...]
    h1 = jnp.maximum(pre, 0.0)
    out_ref[...] = dinv * jnp.dot(
        h1, w_ref[...], preferred_element_type=jnp.float32
    )


def _tc_final(hist_ref, agg_ref, hs_ref, b_ref, out_ref):
    dinv = lax.rsqrt(1.0 + hist_ref[0] + hist_ref[1])
    out_ref[...] = dinv * (agg_ref[0] + agg_ref[1] + hs_ref[...]) + b_ref[...]


_f32 = functools.partial(jax.ShapeDtypeStruct, dtype=jnp.float32)


@jax.jit
def kernel(x, edge_index, W1, b1, W2, b2):
    pad_src = jnp.zeros((NW, EPAD - EPW), jnp.int32)
    pad_dst = jnp.full((NW, EPAD - EPW), N, jnp.int32)
    src = jnp.concatenate(
        [edge_index[0].reshape(NW, EPW), pad_src], axis=1
    ).reshape(NW, NCHUNK, 1, CHUNK)
    dst = jnp.concatenate(
        [edge_index[1].reshape(NW, EPW), pad_dst], axis=1
    ).reshape(NW, NCHUNK, 1, CHUNK)
    idx = jnp.concatenate([src, dst], axis=2)

    hist = _sc_degree(idx).reshape(NC, N, D)[:, :, 0:1]
    h1 = pl.pallas_call(_tc_matmul, out_shape=_f32((N, D)))(x, W1)

    hs1 = pl.pallas_call(_tc_scale, out_shape=_f32((N, D)))(hist, h1)

    agg1 = _sc_aggregate(hs1, idx).reshape(NC, N, D)

    hs2 = pl.pallas_call(_tc_mid, out_shape=_f32((N, D)))(
        hist, agg1, hs1, b1.reshape(1, D), W2
    )

    agg2 = _sc_aggregate(hs2, idx).reshape(NC, N, D)

    out = pl.pallas_call(_tc_final, out_shape=_f32((N, D)))(
        hist, agg2, hs2, b2.reshape(1, D)
    )
    return out


# speedup vs baseline: 1.4674x; 1.4222x over previous
import functools

import jax
import jax.numpy as jnp
from jax import lax
from jax.experimental import pallas as pl
from jax.experimental.pallas import tpu as pltpu
from jax.experimental.pallas import tpu_sc as plsc

N = 10000
D = 128
E = 320000
NC = 2
NS = 16
NW = NC * NS
EPW = E // NW
CHUNK = 128
NCHUNK = 79
EPAD = NCHUNK * CHUNK
NA = N + 8
RPT = N // NS
RZ = 125

_mesh = plsc.VectorSubcoreMesh(
    core_axis_name="c", subcore_axis_name="s", num_cores=NC, num_subcores=NS
)


@functools.partial(
    pl.kernel,
    out_type=jax.ShapeDtypeStruct((NC, NS, RPT, D), jnp.float32),
    mesh=_mesh,
    scratch_types=[
        pltpu.VMEM((2, CHUNK), jnp.int32),
        pltpu.VMEM((2, CHUNK), jnp.int32),
        pltpu.VMEM((2, CHUNK), jnp.int32),
        pltpu.VMEM((2, CHUNK), jnp.int32),
        pltpu.VMEM((CHUNK, D), jnp.float32),
        pltpu.VMEM_SHARED((NA, D), jnp.float32),
        pltpu.SemaphoreType.DMA,
        pltpu.SemaphoreType.DMA,
        pltpu.SemaphoreType.DMA,
        pltpu.SemaphoreType.DMA,
    ],
)
def _sc_degree(dst_hbm, out_hbm, i0, i1, i2, i3, ones_v, acc, s0, s1, s2, s3):
    c = lax.axis_index("c")
    s = lax.axis_index("s")
    wid = c * NS + s
    idxs = (i0, i1, i2, i3)
    ssem = (s0, s1, s2, s3)

    def zrow(i, _):
        for k in range(D // 16):
            ones_v[i, pl.ds(k * 16, 16)] = jnp.zeros((16,), jnp.float32)
        return 0

    lax.fori_loop(0, CHUNK, zrow, 0)
    for r in range(RPT // CHUNK):
        pltpu.sync_copy(ones_v, acc.at[pl.ds(s * RPT + r * CHUNK, CHUNK)])
    rtail = RPT % CHUNK
    pltpu.sync_copy(
        ones_v.at[pl.ds(0, rtail)],
        acc.at[pl.ds(s * RPT + RPT - rtail, rtail)],
    )

    def orow(i, _):
        for k in range(D // 16):
            ones_v[i, pl.ds(k * 16, 16)] = jnp.ones((16,), jnp.float32)
        return 0

    lax.fori_loop(0, CHUNK, orow, 0)
    plsc.subcore_barrier()

    def load_scatter(b, j):
        pltpu.sync_copy(dst_hbm.at[wid, j], idxs[b])
        pltpu.async_copy(ones_v, acc.at[idxs[b].at[1]], ssem[b], add=True)

    def wait_scatter(b):
        pltpu.make_async_copy(ones_v, acc.at[idxs[b].at[1]], ssem[b]).wait()

    for b in range(4):
        load_scatter(b, b)

    def round4(jj, _):
        base = 4 * jj
        for b in range(4):
            wait_scatter(b)
            load_scatter(b, base + b)
        return 0

    lax.fori_loop(1, NCHUNK // 4, round4, 0)
    for t in range(NCHUNK % 4):
        wait_scatter(t)
        load_scatter(t, 4 * (NCHUNK // 4) + t)
    for b in range(4):
        wait_scatter(b)
    plsc.subcore_barrier()
    pltpu.sync_copy(acc.at[pl.ds(s * RPT, RPT)], out_hbm.at[c, s])


@functools.partial(
    pl.kernel,
    out_type=jax.ShapeDtypeStruct((NC, NS, RPT, D), jnp.float32),
    mesh=_mesh,
    scratch_types=[
        pltpu.VMEM((2, CHUNK), jnp.int32),
        pltpu.VMEM((2, CHUNK), jnp.int32),
        pltpu.VMEM((CHUNK, D), jnp.float32),
        pltpu.VMEM((CHUNK, D), jnp.float32),
        pltpu.VMEM_SHARED((NA, D), jnp.float32),
        pltpu.SemaphoreType.DMA,
        pltpu.SemaphoreType.DMA,
    ],
)
def _sc_aggregate(
    table_hbm, idx_hbm, out_hbm, i0, i1, r0, r1, acc, g0, g1
):
    c = lax.axis_index("c")
    s = lax.axis_index("s")
    wid = c * NS + s
    idxs = (i0, i1)
    rows = (r0, r1)
    gsem = (g0, g1)

    def zrow(i, _):
        for k in range(D // 16):
            r0[i, pl.ds(k * 16, 16)] = jnp.zeros((16,), jnp.float32)
        return 0

    lax.fori_loop(0, CHUNK, zrow, 0)
    for r in range(RPT // CHUNK):
        pltpu.sync_copy(r0, acc.at[pl.ds(s * RPT + r * CHUNK, CHUNK)])
    rtail = RPT % CHUNK
    pltpu.sync_copy(
        r0.at[pl.ds(0, rtail)], acc.at[pl.ds(s * RPT + RPT - rtail, rtail)]
    )
    plsc.subcore_barrier()

    def load_gather(b, j):
        pltpu.sync_copy(idx_hbm.at[wid, j], idxs[b])
        pltpu.async_copy(table_hbm.at[idxs[b].at[0]], rows[b], gsem[b])

    def wait_gather(b):
        pltpu.make_async_copy(
            table_hbm.at[idxs[b].at[0]], rows[b], gsem[b]
        ).wait()

    def scatter(b):
        pltpu.sync_copy(rows[b], acc.at[idxs[b].at[1]], add=True)

    load_gather(0, 0)

    def pair(jj, _):
        j1 = 2 * jj + 1
        load_gather(1, j1)
        wait_gather(0)
        scatter(0)
        load_gather(0, j1 + 1)
        wait_gather(1)
        scatter(1)
        return 0

    lax.fori_loop(0, (NCHUNK - 1) // 2, pair, 0)
    wait_gather(0)
    scatter(0)
    plsc.subcore_barrier()
    pltpu.sync_copy(acc.at[pl.ds(s * RPT, RPT)], out_hbm.at[c, s])


def _tc_matmul(x_ref, w_ref, out_ref):
    out_ref[...] = jnp.dot(
        x_ref[...], w_ref[...], preferred_element_type=jnp.float32
    )


def _tc_scale(hist_ref, h_ref, out_ref):
    dinv = lax.rsqrt(1.0 + hist_ref[0] + hist_ref[1])
    out_ref[...] = dinv * h_ref[...]


def _tc_mid(hist_ref, agg_ref, hs_ref, b_ref, w_ref, out_ref):
    dinv = lax.rsqrt(1.0 + hist_ref[0] + hist_ref[1])
    pre = dinv * (agg_ref[0] + agg_ref[1] + hs_ref[...]) + b_ref[...]
    h1 = jnp.maximum(pre, 0.0)
    out_ref[...] = dinv * jnp.dot(
        h1, w_ref[...], preferred_element_type=jnp.float32
    )


def _tc_final(hist_ref, agg_ref, hs_ref, b_ref, out_ref):
    dinv = lax.rsqrt(1.0 + hist_ref[0] + hist_ref[1])
    out_ref[...] = dinv * (agg_ref[0] + agg_ref[1] + hs_ref[...]) + b_ref[...]


_f32 = functools.partial(jax.ShapeDtypeStruct, dtype=jnp.float32)


@jax.jit
def kernel(x, edge_index, W1, b1, W2, b2):
    pad_src = jnp.zeros((NW, EPAD - EPW), jnp.int32)
    pad_dst = jnp.full((NW, EPAD - EPW), N, jnp.int32)
    src = jnp.concatenate(
        [edge_index[0].reshape(NW, EPW), pad_src], axis=1
    ).reshape(NW, NCHUNK, 1, CHUNK)
    dst = jnp.concatenate(
        [edge_index[1].reshape(NW, EPW), pad_dst], axis=1
    ).reshape(NW, NCHUNK, 1, CHUNK)
    idx = jnp.concatenate([src, dst], axis=2)

    hist = _sc_degree(idx).reshape(NC, N, D)[:, :, 0:1]
    h1 = pl.pallas_call(_tc_matmul, out_shape=_f32((N, D)))(x, W1)

    hs1 = pl.pallas_call(_tc_scale, out_shape=_f32((N, D)))(hist, h1)

    agg1 = _sc_aggregate(hs1, idx).reshape(NC, N, D)

    hs2 = pl.pallas_call(_tc_mid, out_shape=_f32((N, D)))(
        hist, agg1, hs1, b1.reshape(1, D), W2
    )

    agg2 = _sc_aggregate(hs2, idx).reshape(NC, N, D)

    out = pl.pallas_call(_tc_final, out_shape=_f32((N, D)))(
        hist, agg2, hs2, b2.reshape(1, D)
    )
    return out
